# Initial kernel scaffold; baseline (speedup 1.0000x reference)
import jax
import jax.numpy as jnp
from jax.experimental import pallas as pl


def kernel(latent, codebook):
    raise NotImplementedError("write your pallas kernel here")


# speedup vs baseline: 1.4163x; 1.4163x over previous
import functools

import jax
import jax.numpy as jnp
from jax.experimental import pallas as pl
from jax.experimental.pallas import tpu as pltpu

_BETA = 0.25
_BC = 1024


def _argmin_body(a_ref, x_ref, cb_ref, b_ref, idx_ref, mval_ref, midx_ref,
                 *, nt, nc, bc):
    j = pl.program_id(0)
    m = jax.lax.dot_general(
        x_ref[...], cb_ref[...], (((1,), (1,)), ((), ())),
        precision=jax.lax.Precision.DEFAULT,
        preferred_element_type=jnp.float32)
    d = (a_ref[...] - 2.0 * m) + b_ref[...]
    tmin = jnp.min(d, axis=1, keepdims=True)
    col = jax.lax.broadcasted_iota(jnp.int32, (nt, bc), 1) + j * bc
    targ = jnp.min(jnp.where(d == tmin, col, nc), axis=1, keepdims=True)

    @pl.when(j == 0)
    def _():
        mval_ref[...] = tmin
        midx_ref[...] = targ

    @pl.when(j != 0)
    def _():
        better = tmin < mval_ref[...]
        mval_ref[...] = jnp.where(better, tmin, mval_ref[...])
        midx_ref[...] = jnp.where(better, targ, midx_ref[...])

    @pl.when(j == nc // bc - 1)
    def _():
        idx_ref[...] = midx_ref[...]


def _gather_body(idx_ref, cb_ref, x_ref, qst_ref, loss_ref, acc_ref,
                 *, nt, nc, bc):
    j = pl.program_id(0)
    col = jax.lax.broadcasted_iota(jnp.int32, (nt, bc), 1) + j * bc
    oh = (idx_ref[...] == col).astype(jnp.float32)
    part = jax.lax.dot_general(
        oh, cb_ref[...], (((1,), (0,)), ((), ())),
        precision=jax.lax.Precision.HIGHEST,
        preferred_element_type=jnp.float32)

    @pl.when(j == 0)
    def _():
        acc_ref[...] = part

    @pl.when(j != 0)
    def _():
        acc_ref[...] += part

    @pl.when(j == nc // bc - 1)
    def _():
        x = x_ref[...]
        q = acc_ref[...]
        qst_ref[...] = x + (q - x)
        diff = x - q
        msq = jnp.mean(diff * diff)
        loss_ref[...] = jnp.full((1, 1), _BETA * msq + msq, jnp.float32)


def kernel(latent, codebook):
    B, S, D = latent.shape
    nt = B * S
    nc = codebook.shape[0]
    bc = _BC
    flat = latent.reshape(-1, D)
    a = jnp.sum(flat ** 2, axis=1, keepdims=True)
    b = jnp.sum(codebook ** 2, axis=1).reshape(1, nc)
    grid = (nc // bc,)

    idx = pl.pallas_call(
        functools.partial(_argmin_body, nt=nt, nc=nc, bc=bc),
        grid=grid,
        in_specs=[
            pl.BlockSpec((nt, 1), lambda j: (0, 0)),
            pl.BlockSpec((nt, D), lambda j: (0, 0)),
            pl.BlockSpec((bc, D), lambda j: (j, 0)),
            pl.BlockSpec((1, bc), lambda j: (0, j)),
        ],
        out_specs=pl.BlockSpec((nt, 1), lambda j: (0, 0)),
        out_shape=jax.ShapeDtypeStruct((nt, 1), jnp.int32),
        scratch_shapes=[pltpu.VMEM((nt, 1), jnp.float32),
                        pltpu.VMEM((nt, 1), jnp.int32)],
    )(a, flat, codebook, b)

    qst, loss = pl.pallas_call(
        functools.partial(_gather_body, nt=nt, nc=nc, bc=bc),
        grid=grid,
        in_specs=[
            pl.BlockSpec((nt, 1), lambda j: (0, 0)),
            pl.BlockSpec((bc, D), lambda j: (j, 0)),
            pl.BlockSpec((nt, D), lambda j: (0, 0)),
        ],
        out_specs=[pl.BlockSpec((nt, D), lambda j: (0, 0)),
                   pl.BlockSpec((1, 1), lambda j: (0, 0))],
        out_shape=[jax.ShapeDtypeStruct((nt, D), jnp.float32),
                   jax.ShapeDtypeStruct((1, 1), jnp.float32)],
        scratch_shapes=[pltpu.VMEM((nt, D), jnp.float32)],
    )(idx, codebook, flat)

    return (qst.reshape(B, S, D), loss.reshape(()), idx.reshape(nt))


# speedup vs baseline: 3.4125x; 2.4094x over previous
import functools

import jax
import jax.numpy as jnp
from jax.experimental import pallas as pl
from jax.experimental.pallas import tpu as pltpu
from jax.experimental.pallas import tpu_sc as plsc

_BETA = 0.25
_BC = 1024
_GW = 128


def _argmin_body(a_ref, x_ref, cb_ref, b_ref, idx_ref, rmin_ref, raux_ref,
                 *, nt, nc, bc):
    j = pl.program_id(0)
    ng = bc // 128

    @pl.when(j == 0)
    def _():
        rmin_ref[...] = jnp.full((nt, 128), jnp.inf, jnp.float32)
        raux_ref[...] = jnp.zeros((nt, 128), jnp.int32)

    m = jax.lax.dot_general(
        x_ref[...], cb_ref[...], (((1,), (1,)), ((), ())),
        precision=jax.lax.Precision.DEFAULT,
        preferred_element_type=jnp.float32)
    a = a_ref[...]
    for g in range(ng):
        sl = slice(g * 128, (g + 1) * 128)
        d = (a - 2.0 * m[:, sl]) + b_ref[:, sl]
        pred = d < rmin_ref[...]
        rmin_ref[...] = jnp.where(pred, d, rmin_ref[...])
        raux_ref[...] = jnp.where(pred, j * ng + g, raux_ref[...])

    @pl.when(j == nc // bc - 1)
    def _():
        rmin = rmin_ref[...]
        gmin = jnp.min(rmin, axis=1, keepdims=True)
        col = (raux_ref[...] * 128
               + jax.lax.broadcasted_iota(jnp.int32, (nt, 128), 1))
        cand = jnp.where(rmin == gmin, col, nc)
        idx_ref[...] = jnp.min(cand, axis=1, keepdims=True)


def _sc_gather(cb_pad, idx_row, nt):
    mesh = plsc.VectorSubcoreMesh(core_axis_name="core",
                                  subcore_axis_name="subcore")
    w = cb_pad.shape[1]

    @pl.kernel(out_type=jax.ShapeDtypeStruct((nt, w), cb_pad.dtype),
               mesh=mesh)
    def gk(cb_hbm, i_hbm, o_hbm):
        def body(i_vmem, o_vmem):
            pltpu.sync_copy(cb_hbm.at[i_vmem.at[0]], o_vmem)

        pltpu.emit_pipeline(
            body,
            grid=(nt // _GW,),
            in_specs=[pl.BlockSpec((1, _GW), index_map=lambda i: (0, i))],
            out_specs=[pl.BlockSpec((_GW, w), index_map=lambda i: (i, 0))],
            core_axis_name=("core", "subcore"),
            dimension_semantics=(pltpu.PARALLEL,),
        )(i_hbm, o_hbm)

    return gk(cb_pad, idx_row)


def _finish_body(x_ref, q_ref, qst_ref, loss_ref):
    x = x_ref[...]
    q = q_ref[:, :x_ref.shape[1]]
    qst_ref[...] = x + (q - x)
    diff = x - q
    msq = jnp.mean(diff * diff)
    loss_ref[...] = jnp.full((1, 1), _BETA * msq + msq, jnp.float32)


def kernel(latent, codebook):
    B, S, D = latent.shape
    nt = B * S
    nc = codebook.shape[0]
    bc = _BC
    flat = latent.reshape(-1, D)
    a = jnp.sum(flat ** 2, axis=1, keepdims=True)
    b = jnp.sum(codebook ** 2, axis=1).reshape(1, nc)
    grid = (nc // bc,)

    idx = pl.pallas_call(
        functools.partial(_argmin_body, nt=nt, nc=nc, bc=bc),
        grid=grid,
        in_specs=[
            pl.BlockSpec((nt, 1), lambda j: (0, 0)),
            pl.BlockSpec((nt, D), lambda j: (0, 0)),
            pl.BlockSpec((bc, D), lambda j: (j, 0)),
            pl.BlockSpec((1, bc), lambda j: (0, j)),
        ],
        out_specs=pl.BlockSpec((nt, 1), lambda j: (0, 0)),
        out_shape=jax.ShapeDtypeStruct((nt, 1), jnp.int32),
        scratch_shapes=[pltpu.VMEM((nt, 128), jnp.float32),
                        pltpu.VMEM((nt, 128), jnp.int32)],
    )(a, flat, codebook, b)

    cb_pad = jnp.pad(codebook, ((0, 0), (0, 128 - D)))
    q = _sc_gather(cb_pad, idx.reshape(1, nt), nt)

    qst, loss = pl.pallas_call(
        _finish_body,
        in_specs=[
            pl.BlockSpec((nt, D), lambda: (0, 0)),
            pl.BlockSpec((nt, 128), lambda: (0, 0)),
        ],
        out_specs=[pl.BlockSpec((nt, D), lambda: (0, 0)),
                   pl.BlockSpec((1, 1), lambda: (0, 0))],
        out_shape=[jax.ShapeDtypeStruct((nt, D), jnp.float32),
                   jax.ShapeDtypeStruct((1, 1), jnp.float32)],
    )(flat, q)

    return (qst.reshape(B, S, D), loss.reshape(()), idx.reshape(nt))
